# Initial kernel scaffold; baseline (speedup 1.0000x reference)
import jax
import jax.numpy as jnp
from jax.experimental import pallas as pl


def kernel(logits):
    raise NotImplementedError("write your pallas kernel here")


# speedup vs baseline: 7.3059x; 7.3059x over previous
import functools

import jax
import jax.numpy as jnp
from jax import lax
from jax.experimental import pallas as pl
from jax.experimental.pallas import tpu as pltpu
from jax.experimental.pallas import tpu_sc as plsc

N_ROWS = 32768
N_COLS = 64
K = 8
NC = 2
NS = 16
L = 16
NW = NC * NS
RPW = N_ROWS // NW
GROUPS = RPW // L


def _sc_body(logits_hbm, idx_hbm, prob_hbm, vals_v, idx_v, val_v, prob_v):
    wid = lax.axis_index("s") * NC + lax.axis_index("c")
    pltpu.sync_copy(logits_hbm.at[pl.ds(wid * (RPW * N_COLS), RPW * N_COLS)],
                    vals_v)

    lane = lax.iota(jnp.int32, L)
    neg = jnp.full((L,), -jnp.inf, jnp.float32)
    cint = [jnp.full((L,), j, jnp.int32) for j in range(N_COLS)]

    def group(g, carry):
        vbase = (g * L + lane) * N_COLS
        obase = (g * L + lane) * K

        regs = [neg] * K
        for j in range(N_COLS):
            x = plsc.load_gather(vals_v, [vbase + cint[j]])
            gt = [x > r for r in regs]
            new_regs = []
            for i in range(K):
                shifted = jnp.where(gt[i + 1], regs[i + 1], x) if i + 1 < K else x
                new_regs.append(jnp.where(gt[i], shifted, regs[i]))
            regs = new_regs
        t = regs[0]
        m = regs[K - 1]
        eq_budget = jnp.zeros((L,), jnp.int32)
        for r in regs:
            eq_budget = eq_budget + jnp.where(r == t, 1, 0)

        eq_seen = jnp.zeros((L,), jnp.int32)
        cnt = jnp.zeros((L,), jnp.int32)
        for j in range(N_COLS):
            x = plsc.load_gather(vals_v, [vbase + cint[j]])
            is_eq = x == t
            sel = jnp.logical_or(x > t,
                                 jnp.logical_and(is_eq, eq_seen < eq_budget))
            pos = obase + jnp.minimum(cnt, K - 1)
            plsc.store_scatter(idx_v, [pos], cint[j], mask=sel)
            plsc.store_scatter(val_v, [pos], x, mask=sel)
            cnt = cnt + jnp.where(sel, 1, 0)
            eq_seen = eq_seen + jnp.where(is_eq, 1, 0)

        es = []
        denom = jnp.zeros((L,), jnp.float32)
        for p in range(K):
            vp = plsc.load_gather(val_v, [obase + cint[p]])
            e = jnp.exp(vp - m)
            es.append(e)
            denom = denom + e
        inv = 1.0 / denom
        for p in range(K):
            plsc.store_scatter(prob_v, [obase + cint[p]], es[p] * inv)
        return carry

    lax.fori_loop(0, GROUPS, group, jnp.int32(0))

    pltpu.sync_copy(idx_v, idx_hbm.at[pl.ds(wid * (RPW * K), RPW * K)])
    pltpu.sync_copy(prob_v, prob_hbm.at[pl.ds(wid * (RPW * K), RPW * K)])


_sc_call = functools.partial(
    pl.kernel,
    out_type=(
        jax.ShapeDtypeStruct((N_ROWS * K,), jnp.int32),
        jax.ShapeDtypeStruct((N_ROWS * K,), jnp.float32),
    ),
    mesh=plsc.VectorSubcoreMesh(
        core_axis_name="c", subcore_axis_name="s",
        num_cores=NC, num_subcores=NS,
    ),
    compiler_params=pltpu.CompilerParams(needs_layout_passes=False),
    scratch_types=[
        pltpu.VMEM((RPW * N_COLS,), jnp.float32),
        pltpu.VMEM((RPW * K,), jnp.int32),
        pltpu.VMEM((RPW * K,), jnp.float32),
        pltpu.VMEM((RPW * K,), jnp.float32),
    ],
)(_sc_body)


def kernel(logits):
    flat_idx, flat_prob = _sc_call(logits.reshape(-1))
    return flat_idx.reshape(N_ROWS, K), flat_prob.reshape(N_ROWS, K)
